# Initial kernel scaffold; baseline (speedup 1.0000x reference)
import jax
import jax.numpy as jnp
from jax.experimental import pallas as pl


def kernel(x, edge_index, emb, Wl0, bl0, Wr0, Wl1, bl1, Wr1, Wl2, bl2, Wr2, W_last, b_last):
    raise NotImplementedError("write your pallas kernel here")


# speedup vs baseline: 3.9198x; 3.9198x over previous
import functools

import jax
import jax.numpy as jnp
from jax import lax
from jax.experimental import pallas as pl
from jax.experimental.pallas import tpu as pltpu
from jax.experimental.pallas import tpu_sc as plsc

NC = 2
NS = 16
NW = NC * NS

N = 10000
NP = 10240
E = 320000
D = 128
V = 1000

CHUNK = 80
EPT = E // NW
NCHUNKS = EPT // CHUNK
RPT = NP // NS
GPT = NP // NW
DEGW = 128


def _mesh():
    return plsc.VectorSubcoreMesh(
        core_axis_name="c", subcore_axis_name="s", num_cores=NC, num_subcores=NS
    )


def _emb_deg_body(emb_h, x_h, dst_h, z16_h, ones_h,
                  h_out, deg_out,
                  idx_v, rows_v, ones_v, z16_v, sem, deg_sh):
    cid = lax.axis_index("c")
    sid = lax.axis_index("s")
    wid = sid * NC + cid

    pltpu.sync_copy(z16_h, z16_v)
    for k in range(RPT // 128):
        pltpu.sync_copy(z16_v, deg_sh.at[pl.ds(sid * RPT + k * 128, 128)])
    plsc.subcore_barrier()

    pltpu.sync_copy(ones_h, ones_v)

    for c4 in range(GPT // CHUNK):
        r0 = wid * GPT + c4 * CHUNK
        pltpu.sync_copy(x_h.at[pl.ds(r0, CHUNK)], idx_v)
        pltpu.async_copy(emb_h.at[idx_v], rows_v, sem).wait()
        pltpu.sync_copy(rows_v, h_out.at[pl.ds(r0, CHUNK)])

    def deg_step(c, carry):
        base = pl.multiple_of(wid * EPT + c * CHUNK, 8)
        pltpu.sync_copy(dst_h.at[pl.ds(base, CHUNK)], idx_v)
        pltpu.sync_copy(ones_v, deg_sh.at[idx_v], add=True)
        return carry

    lax.fori_loop(0, NCHUNKS, deg_step, 0)
    plsc.subcore_barrier()

    pltpu.sync_copy(deg_sh.at[pl.ds(sid * RPT, RPT)],
                    deg_out.at[pl.ds(cid * NP + sid * RPT, RPT)])


def _seg_sum_body(h_h, src_h, dst_h, z_h,
                  p_out,
                  sidx_v, didx_v, rows_v, z_v, sem, agg_sh):
    cid = lax.axis_index("c")
    sid = lax.axis_index("s")
    wid = sid * NC + cid

    pltpu.sync_copy(z_h, z_v)
    for k in range(RPT // 128):
        pltpu.sync_copy(z_v, agg_sh.at[pl.ds(sid * RPT + k * 128, 128)])
    plsc.subcore_barrier()

    def edge_step(c, carry):
        base = pl.multiple_of(wid * EPT + c * CHUNK, 8)
        pltpu.sync_copy(src_h.at[pl.ds(base, CHUNK)], sidx_v)
        pltpu.async_copy(h_h.at[sidx_v], rows_v, sem).wait()
        pltpu.sync_copy(dst_h.at[pl.ds(base, CHUNK)], didx_v)
        pltpu.sync_copy(rows_v, agg_sh.at[didx_v], add=True)
        return carry

    lax.fori_loop(0, NCHUNKS, edge_step, 0)
    plsc.subcore_barrier()

    pltpu.sync_copy(agg_sh.at[pl.ds(sid * RPT, RPT)],
                    p_out.at[pl.ds(cid * NP + sid * RPT, RPT)])


def _layer_body(p_ref, deg_ref, h_ref, wl_ref, wr_ref, bl_ref, o_ref):
    deg = deg_ref[0] + deg_ref[1]
    agg = (p_ref[0] + p_ref[1]) / jnp.maximum(deg, 1.0)
    t = jnp.dot(agg, wl_ref[...], preferred_element_type=jnp.float32)
    t = t + jnp.dot(h_ref[...], wr_ref[...], preferred_element_type=jnp.float32)
    o_ref[...] = jnp.maximum(t + bl_ref[...], 0.0)


def _layer_tc(p, deg, h, wlT, wrT, bl):
    bn = 1024
    grid = (NP // bn,)
    return pl.pallas_call(
        _layer_body,
        grid=grid,
        in_specs=[
            pl.BlockSpec((NC, bn, D), lambda i: (0, i, 0)),
            pl.BlockSpec((NC, bn, 1), lambda i: (0, i, 0)),
            pl.BlockSpec((bn, D), lambda i: (i, 0)),
            pl.BlockSpec((D, D), lambda i: (0, 0)),
            pl.BlockSpec((D, D), lambda i: (0, 0)),
            pl.BlockSpec((1, D), lambda i: (0, 0)),
        ],
        out_specs=pl.BlockSpec((bn, D), lambda i: (i, 0)),
        out_shape=jax.ShapeDtypeStruct((NP, D), jnp.float32),
    )(p, deg, h, wlT, wrT, bl)


def _final_body(h_ref, wt_ref, b_ref, o_ref):
    logits = jnp.dot(h_ref[...], wt_ref[...],
                     preferred_element_type=jnp.float32) + b_ref[...]
    m = jnp.max(logits, axis=1, keepdims=True)
    e = jnp.exp(logits - m)
    o_ref[...] = e / jnp.sum(e, axis=1, keepdims=True)


def _final_tc(h, wT, b):
    bn = 1000
    grid = (N // bn,)
    return pl.pallas_call(
        _final_body,
        grid=grid,
        in_specs=[
            pl.BlockSpec((bn, D), lambda i: (i, 0)),
            pl.BlockSpec((D, V), lambda i: (0, 0)),
            pl.BlockSpec((1, V), lambda i: (0, 0)),
        ],
        out_specs=pl.BlockSpec((bn, V), lambda i: (i, 0)),
        out_shape=jax.ShapeDtypeStruct((N, V), jnp.float32),
    )(h, wT, b)


def kernel(x, edge_index, emb, Wl0, bl0, Wr0, Wl1, bl1, Wr1, Wl2, bl2, Wr2,
           W_last, b_last):
    src = edge_index[0]
    dst = edge_index[1]
    x_p = jnp.concatenate([x, jnp.zeros((NP - N,), x.dtype)])

    z128 = jnp.zeros((128, D), jnp.float32)
    ones128 = jnp.ones((CHUNK, DEGW), jnp.float32)

    emb_deg = pl.kernel(
        _emb_deg_body,
        out_type=[
            jax.ShapeDtypeStruct((NP, D), jnp.float32),
            jax.ShapeDtypeStruct((NC * NP, DEGW), jnp.float32),
        ],
        mesh=_mesh(),
        scratch_types=[
            pltpu.VMEM((CHUNK,), jnp.int32),
            pltpu.VMEM((CHUNK, D), jnp.float32),
            pltpu.VMEM((CHUNK, DEGW), jnp.float32),
            pltpu.VMEM((128, DEGW), jnp.float32),
            pltpu.SemaphoreType.DMA,
            pltpu.VMEM_SHARED((NP, DEGW), jnp.float32),
        ],
    )
    h, deg = emb_deg(emb, x_p, dst, z128, ones128)
    deg = deg.reshape(NC, NP, DEGW)[:, :, 0:1]

    seg_sum = pl.kernel(
        _seg_sum_body,
        out_type=jax.ShapeDtypeStruct((NC * NP, D), jnp.float32),
        mesh=_mesh(),
        scratch_types=[
            pltpu.VMEM((CHUNK,), jnp.int32),
            pltpu.VMEM((CHUNK,), jnp.int32),
            pltpu.VMEM((CHUNK, D), jnp.float32),
            pltpu.VMEM((128, D), jnp.float32),
            pltpu.SemaphoreType.DMA,
            pltpu.VMEM_SHARED((NP, D), jnp.float32),
        ],
    )

    for Wl, bl, Wr in ((Wl0, bl0, Wr0), (Wl1, bl1, Wr1), (Wl2, bl2, Wr2)):
        p = seg_sum(h, src, dst, z128).reshape(NC, NP, D)
        h = _layer_tc(p, deg, h, Wl.T, Wr.T, bl.reshape(1, D))

    return _final_tc(h, W_last.T, b_last.reshape(1, V))
